# baseline (speedup 1.0000x reference)
import functools

import jax
import jax.numpy as jnp
from jax import lax
from jax.experimental import pallas as pl
from jax.experimental.pallas import tpu as pltpu
from jax.experimental.pallas import tpu_sc as plsc

_D = 256
_K = 8192
_B = 8192
_BN = 2048
_BK = 2048

_NC, _NS = 2, 16
_NW = _NC * _NS
_BPW = _B // _NW


def _dist_body(z_ref, w_ref, z2_ref, w2_ref, idx_ref, lsum_ref, t2_ref):
    i = pl.program_id(0)
    z = z_ref[...].astype(jnp.bfloat16)
    z2 = z2_ref[...]
    nj = _K // _BK
    sio = lax.broadcasted_iota(jnp.int32, (8, _BN), 0).astype(jnp.float32)

    def body(j, carry):
        run_val, run_idx, run_true = carry
        wb = w_ref[pl.ds(j * _BK, _BK), :]
        t2_ref[...] = lax.dot_general(wb, z, (((1,), (1,)), ((), ())),
                                      preferred_element_type=jnp.float32)

        def qbody(q, qc):
            qval, qrow = qc
            tq = t2_ref[pl.ds(q * 8, 8), :]
            w2q = w2_ref[pl.ds(j * _BK + q * 8, 8), :]
            dq = (z2 + tq) + w2q
            win = dq < qval
            qf = q.astype(jnp.float32)
            return (jnp.where(win, dq, qval), jnp.where(win, qf, qrow))

        qval, qrow = lax.fori_loop(
            0, _BK // 8, qbody,
            (jnp.full((8, _BN), jnp.inf, jnp.float32),
             jnp.zeros((8, _BN), jnp.float32)), unroll=8)
        jrow = qrow * 8.0 + sio
        lv = jnp.min(qval, axis=0)
        lif = jnp.min(jnp.where(qval == lv[None, :], jrow, jnp.float32(1e9)),
                      axis=0)
        li = lif.astype(jnp.int32) + (j * _BK)
        better = lv < run_val
        lq = lv.astype(jnp.bfloat16).astype(jnp.float32)
        return (jnp.where(better, lq, run_val),
                jnp.where(better, li, run_idx),
                jnp.where(better, lv, run_true))

    rv, ri, rt = lax.fori_loop(
        0, nj, body,
        (jnp.full((_BN,), jnp.inf, jnp.float32),
         jnp.zeros((_BN,), jnp.int32),
         jnp.full((_BN,), jnp.inf, jnp.float32)))
    idx_ref[...] = ri[None, None, :]

    part = jnp.sum(rt).reshape(1, 1)
    prev = jnp.where(i == 0, jnp.zeros((1, 1), jnp.float32), lsum_ref[...])
    lsum_ref[...] = prev + part


def _nearest(z, w_s, z2, w2):
    n = z.shape[0]
    return pl.pallas_call(
        _dist_body,
        grid=(n // _BN,),
        in_specs=[
            pl.BlockSpec((_BN, _D), lambda i: (i, 0)),
            pl.BlockSpec((_K, _D), lambda i: (0, 0)),
            pl.BlockSpec((1, _BN), lambda i: (0, i)),
            pl.BlockSpec((_K, 1), lambda i: (0, 0)),
        ],
        out_specs=[
            pl.BlockSpec((1, 1, _BN), lambda i: (i, 0, 0)),
            pl.BlockSpec((1, 1), lambda i: (0, 0)),
        ],
        out_shape=[
            jax.ShapeDtypeStruct((n // _BN, 1, _BN), jnp.int32),
            jax.ShapeDtypeStruct((1, 1), jnp.float32),
        ],
        scratch_shapes=[pltpu.VMEM((_BK, _BN), jnp.float32)],
    )(z, w_s, z2, w2)


def _gather_body(table_hbm, idx_hbm, out_hbm, idx_v, rows_v, sem):
    wid = lax.axis_index("s") * _NC + lax.axis_index("c")
    nb = idx_hbm.shape[0] // _NW
    base = wid * nb
    pltpu.sync_copy(idx_hbm.at[pl.ds(base, nb)], idx_v)
    pltpu.async_copy(table_hbm.at[idx_v], rows_v, sem).wait()
    pltpu.sync_copy(rows_v, out_hbm.at[pl.ds(base, nb)])


def _gather(W, idx):
    nb = idx.shape[0] // _NW
    k = functools.partial(
        pl.kernel,
        out_type=jax.ShapeDtypeStruct((idx.shape[0], _D), jnp.float32),
        mesh=plsc.VectorSubcoreMesh(core_axis_name="c", subcore_axis_name="s"),
        scratch_types=[
            pltpu.VMEM((nb,), jnp.int32),
            pltpu.VMEM((nb, _D), jnp.float32),
            pltpu.SemaphoreType.DMA,
        ],
    )(_gather_body)
    return k(W, idx)


def kernel(x, W):
    z = x.reshape(-1, _D)
    z2 = jnp.sum(z * z, axis=1, keepdims=True)
    w2 = jnp.sum(W * W, axis=1)[None, :]
    idx2, lsum = _nearest(z, (W * -2.0).astype(jnp.bfloat16),
                          z2.reshape(1, -1), w2.reshape(-1, 1))
    zq = _gather(W, idx2.reshape(-1))
    loss = lsum[0, 0] / jnp.float32(z.size)
    return (zq.reshape(x.shape), loss, loss,
            idx2.reshape(x.shape[:-1]))


# speedup vs baseline: 1.3596x; 1.3596x over previous
import functools

import jax
import jax.numpy as jnp
from jax import lax
from jax.experimental import pallas as pl
from jax.experimental.pallas import tpu as pltpu
from jax.experimental.pallas import tpu_sc as plsc

_D = 256
_K = 8192
_B = 8192
_BN = 2048
_BK = 2048

_NC, _NS = 2, 16
_NW = _NC * _NS
_BPW = _B // _NW


def _dist_body(z_ref, w_ref, z2_ref, w2_ref, idx_ref, lsum_ref):
    i = pl.program_id(0)
    z = z_ref[...].astype(jnp.bfloat16)
    z2 = z2_ref[...]
    nj = _K // _BK
    ii = lax.broadcasted_iota(jnp.int32, (_BK, _BN), 0).astype(jnp.float32)

    def body(j, carry):
        run_val, run_idx, run_true = carry
        wb = w_ref[pl.ds(j * _BK, _BK), :]
        t2 = lax.dot_general(wb, z, (((1,), (1,)), ((), ())),
                             preferred_element_type=jnp.float32)
        w2 = w2_ref[pl.ds(j * _BK, _BK), :]
        d = (z2 + t2) + w2
        lv = jnp.min(d, axis=0)
        lif = jnp.min(jnp.where(d == lv[None, :], ii, jnp.float32(1e9)),
                      axis=0)
        li = lif.astype(jnp.int32) + (j * _BK)
        better = lv < run_val
        lq = lv.astype(jnp.bfloat16).astype(jnp.float32)
        return (jnp.where(better, lq, run_val),
                jnp.where(better, li, run_idx),
                jnp.where(better, lv, run_true))

    rv, ri, rt = lax.fori_loop(
        0, nj, body,
        (jnp.full((_BN,), jnp.inf, jnp.float32),
         jnp.zeros((_BN,), jnp.int32),
         jnp.full((_BN,), jnp.inf, jnp.float32)))
    idx_ref[...] = ri[None, None, :]

    part = jnp.sum(rt).reshape(1, 1)
    prev = jnp.where(i == 0, jnp.zeros((1, 1), jnp.float32), lsum_ref[...])
    lsum_ref[...] = prev + part


def _nearest(z, w_s, z2, w2):
    n = z.shape[0]
    return pl.pallas_call(
        _dist_body,
        grid=(n // _BN,),
        in_specs=[
            pl.BlockSpec((_BN, _D), lambda i: (i, 0)),
            pl.BlockSpec((_K, _D), lambda i: (0, 0)),
            pl.BlockSpec((1, _BN), lambda i: (0, i)),
            pl.BlockSpec((_K, 1), lambda i: (0, 0)),
        ],
        out_specs=[
            pl.BlockSpec((1, 1, _BN), lambda i: (i, 0, 0)),
            pl.BlockSpec((1, 1), lambda i: (0, 0)),
        ],
        out_shape=[
            jax.ShapeDtypeStruct((n // _BN, 1, _BN), jnp.int32),
            jax.ShapeDtypeStruct((1, 1), jnp.float32),
        ],
    )(z, w_s, z2, w2)


def _gather_body(table_hbm, idx_hbm, out_hbm, idx_v, rows_v, sem):
    wid = lax.axis_index("s") * _NC + lax.axis_index("c")
    nb = idx_hbm.shape[0] // _NW
    base = wid * nb
    pltpu.sync_copy(idx_hbm.at[pl.ds(base, nb)], idx_v)
    pltpu.async_copy(table_hbm.at[idx_v], rows_v, sem).wait()
    pltpu.sync_copy(rows_v, out_hbm.at[pl.ds(base, nb)])


def _gather(W, idx):
    nb = idx.shape[0] // _NW
    k = functools.partial(
        pl.kernel,
        out_type=jax.ShapeDtypeStruct((idx.shape[0], _D), jnp.float32),
        mesh=plsc.VectorSubcoreMesh(core_axis_name="c", subcore_axis_name="s"),
        scratch_types=[
            pltpu.VMEM((nb,), jnp.int32),
            pltpu.VMEM((nb, _D), jnp.float32),
            pltpu.SemaphoreType.DMA,
        ],
    )(_gather_body)
    return k(W, idx)


def kernel(x, W):
    z = x.reshape(-1, _D)
    z2 = jnp.sum(z * z, axis=1, keepdims=True)
    w2 = jnp.sum(W * W, axis=1)[None, :]
    idx2, lsum = _nearest(z, (W * -2.0).astype(jnp.bfloat16),
                          z2.reshape(1, -1), w2.reshape(-1, 1))
    zq = _gather(W, idx2.reshape(-1))
    loss = lsum[0, 0] / jnp.float32(z.size)
    return (zq.reshape(x.shape), loss, loss,
            idx2.reshape(x.shape[:-1]))
